# baseline (speedup 1.0000x reference)
import functools

import jax
import jax.numpy as jnp
from jax import lax
from jax.experimental import pallas as pl
from jax.experimental.pallas import tpu as pltpu
from jax.experimental.pallas import tpu_sc as plsc

N = 10000
E = 320000
D = 128
V = 1000
C = 3

NP = 10240

NC = 2
NS = 16
NW = NC * NS
EPW = E // NW
CHUNK = 40
NCH = EPW // CHUNK
ROWS_PER_TILE = NP // NS

LPW = NP // NW
LCHUNK = 80
NLCHUNK = LPW // LCHUNK

NR = 8
NI = 14
LG = 6
LI = 12

_mesh = plsc.VectorSubcoreMesh(core_axis_name="c", subcore_axis_name="s")


@functools.partial(
    pl.kernel,
    out_type=jax.ShapeDtypeStruct((NP, D), jnp.float32),
    mesh=_mesh,
    scratch_types=[
        pltpu.VMEM((LPW,), jnp.int32),
        pltpu.VMEM((LPW, D), jnp.float32),
        pltpu.SemaphoreType.DMA,
    ],
)
def _sc_embed(emb_hbm, idx_hbm, out_hbm, idx_v, rows_v, sem):
    c = lax.axis_index("c")
    s = lax.axis_index("s")
    wid = s * NC + c
    base = wid * LPW
    pltpu.sync_copy(idx_hbm.at[pl.ds(base, LPW)], idx_v)
    for j in range(NLCHUNK):
        pltpu.async_copy(emb_hbm.at[idx_v.at[pl.ds(j * LCHUNK, LCHUNK)]],
                         rows_v.at[pl.ds(j * LCHUNK, LCHUNK)], sem)
    for j in range(NLCHUNK):
        pltpu.make_async_copy(emb_hbm.at[idx_v.at[pl.ds(j * LCHUNK, LCHUNK)]],
                              rows_v.at[pl.ds(j * LCHUNK, LCHUNK)], sem).wait()
    pltpu.sync_copy(rows_v, out_hbm.at[pl.ds(base, LPW)])


@functools.partial(
    pl.kernel,
    out_type=jax.ShapeDtypeStruct((NC, NP, D), jnp.float32),
    mesh=_mesh,
    scratch_types=[
        pltpu.VMEM((NI, CHUNK), jnp.int32),
        pltpu.VMEM((NI, CHUNK), jnp.int32),
        pltpu.VMEM((NR * CHUNK, D), jnp.float32),
        pltpu.SemaphoreType.DMA((NI,)),
        pltpu.SemaphoreType.DMA((NR,)),
        pltpu.SemaphoreType.DMA((NR,)),
        pltpu.VMEM_SHARED((NP, D), jnp.float32),
    ],
)
def _sc_agg(h_hbm, src_hbm, dst_hbm, zero_hbm, out_hbm,
            src_t, dst_t, rows_v, isem, gsem, ssem, agg_sh):
    c = lax.axis_index("c")
    s = lax.axis_index("s")
    wid = s * NC + c
    r0 = s * ROWS_PER_TILE
    pltpu.sync_copy(zero_hbm.at[pl.ds(r0, ROWS_PER_TILE)],
                    agg_sh.at[pl.ds(r0, ROWS_PER_TILE)])
    plsc.subcore_barrier()

    def rows_slot(r):
        return rows_v.at[pl.ds(r * CHUNK, CHUNK)]

    def fire_idx(ch):
        i = ch % NI
        pltpu.async_copy(src_hbm.at[wid, ch], src_t.at[i], isem.at[i])
        pltpu.async_copy(dst_hbm.at[wid, ch], dst_t.at[i], isem.at[i])

    def wait_idx(ch):
        i = ch % NI
        pltpu.make_async_copy(src_hbm.at[wid, ch], src_t.at[i],
                              isem.at[i]).wait()
        pltpu.make_async_copy(dst_hbm.at[wid, ch], dst_t.at[i],
                              isem.at[i]).wait()

    def fire_gather(ch):
        pltpu.async_copy(h_hbm.at[src_t.at[ch % NI]], rows_slot(ch % NR),
                         gsem.at[ch % NR])

    def wait_gather(ch):
        pltpu.make_async_copy(h_hbm.at[src_t.at[ch % NI]],
                              rows_slot(ch % NR), gsem.at[ch % NR]).wait()

    def fire_scatter(ch):
        pltpu.async_copy(rows_slot(ch % NR), agg_sh.at[dst_t.at[ch % NI]],
                         ssem.at[ch % NR], add=True)

    def wait_scatter(ch):
        pltpu.make_async_copy(rows_slot(ch % NR),
                              agg_sh.at[dst_t.at[ch % NI]],
                              ssem.at[ch % NR]).wait()

    for k in range(LI):
        fire_idx(k)
    for k in range(LG):
        wait_idx(k)
        fire_gather(k)

    @pl.loop(0, NCH)
    def _(ch):
        @pl.when(ch >= NR - LG)
        def _():
            wait_scatter(ch - (NR - LG))

        @pl.when(ch + LI < NCH)
        def _():
            fire_idx(ch + LI)

        @pl.when(ch + LG < NCH)
        def _():
            wait_idx(ch + LG)
            fire_gather(ch + LG)

        wait_gather(ch)
        fire_scatter(ch)

    for k in range(NCH - (NR - LG), NCH):
        wait_scatter(k)

    plsc.subcore_barrier()
    pltpu.sync_copy(agg_sh.at[pl.ds(r0, ROWS_PER_TILE)],
                    out_hbm.at[c, pl.ds(r0, ROWS_PER_TILE)])


BN = 2048


def _tc_layer_body(agg_ref, w_ref, b_ref, out_ref):
    x = agg_ref[0] + agg_ref[1]
    y = jnp.dot(x, w_ref[...], preferred_element_type=jnp.float32)
    out_ref[...] = jnp.maximum(y + b_ref[...], 0.0)


def _tc_layer_res_body(agg_ref, w_ref, b_ref, res_ref, out_ref):
    x = agg_ref[0] + agg_ref[1]
    y = jnp.dot(x, w_ref[...], preferred_element_type=jnp.float32)
    out_ref[...] = jnp.maximum(y + b_ref[...], 0.0) + res_ref[...]


def _tc_decode_body(agg_ref, w_ref, b_ref, wd_ref, bd_ref, out_ref):
    x = agg_ref[0] + agg_ref[1]
    h = jnp.maximum(
        jnp.dot(x, w_ref[...], preferred_element_type=jnp.float32) + b_ref[...],
        0.0)
    logits = jnp.dot(h, wd_ref[...], preferred_element_type=jnp.float32)
    logits = logits + bd_ref[...]
    col = lax.broadcasted_iota(jnp.int32, logits.shape, 1)
    mask = col < C
    lm = jnp.where(mask, logits, -1e30)
    m = jnp.max(lm, axis=1, keepdims=True)
    e = jnp.exp(lm - m) * mask.astype(jnp.float32)
    out_ref[...] = e / jnp.sum(e, axis=1, keepdims=True)


_agg_spec = pl.BlockSpec((NC, BN, D), lambda i: (0, i, 0))
_w_spec = pl.BlockSpec((D, D), lambda i: (0, 0))
_b_spec = pl.BlockSpec((1, D), lambda i: (0, 0))
_row_spec = pl.BlockSpec((BN, D), lambda i: (i, 0))

_tc_layer = pl.pallas_call(
    _tc_layer_body,
    grid=(NP // BN,),
    in_specs=[_agg_spec, _w_spec, _b_spec],
    out_specs=_row_spec,
    out_shape=jax.ShapeDtypeStruct((NP, D), jnp.float32),
)

_tc_layer_res = pl.pallas_call(
    _tc_layer_res_body,
    grid=(NP // BN,),
    in_specs=[_agg_spec, _w_spec, _b_spec, _row_spec],
    out_specs=_row_spec,
    out_shape=jax.ShapeDtypeStruct((NP, D), jnp.float32),
)

_tc_decode = pl.pallas_call(
    _tc_decode_body,
    grid=(NP // BN,),
    in_specs=[_agg_spec, _w_spec, _b_spec, _w_spec, _b_spec],
    out_specs=_row_spec,
    out_shape=jax.ShapeDtypeStruct((NP, D), jnp.float32),
)


def kernel(labels, edge_index, emb, W1, b1, W2, b2, W3, b3, W4, b4,
           W5, b5, Wd, bd):
    labels = labels.astype(jnp.int32)
    src = edge_index[0].astype(jnp.int32).reshape(NW, NCH, CHUNK)
    dst = edge_index[1].astype(jnp.int32).reshape(NW, NCH, CHUNK)
    labels_pad = jnp.pad(labels, (0, NP - N))
    zeros = jnp.zeros((NP, D), jnp.float32)
    b1r, b2r, b3r, b4r, b5r = (b.reshape(1, D) for b in (b1, b2, b3, b4, b5))
    wd_pad = jnp.zeros((D, D), jnp.float32).at[:, :C].set(Wd)
    bd_pad = jnp.zeros((1, D), jnp.float32).at[0, :C].set(bd)

    h0 = _sc_embed(emb, labels_pad)
    agg = _sc_agg(h0, src, dst, zeros)
    h1 = _tc_layer(agg, W1, b1r)
    agg = _sc_agg(h1, src, dst, zeros)
    h2 = _tc_layer_res(agg, W2, b2r, h1)
    agg = _sc_agg(h2, src, dst, zeros)
    h3 = _tc_layer(agg, W3, b3r)
    agg = _sc_agg(h3, src, dst, zeros)
    h4 = _tc_layer_res(agg, W4, b4r, h3)
    agg = _sc_agg(h4, src, dst, zeros)
    pred_pad = _tc_decode(agg, W5, b5r, wd_pad, bd_pad)
    return pred_pad[:N, :C]


# speedup vs baseline: 1.1081x; 1.1081x over previous
import functools

import jax
import jax.numpy as jnp
from jax import lax
from jax.experimental import pallas as pl
from jax.experimental.pallas import tpu as pltpu
from jax.experimental.pallas import tpu_sc as plsc

N = 10000
E = 320000
D = 128
V = 1000
C = 3

NP = 10240

NC = 2
NS = 16
NW = NC * NS
EPW = E // NW
CHUNK = 80
NCH = EPW // CHUNK
ROWS_PER_TILE = NP // NS

LPW = NP // NW
LCHUNK = 80
NLCHUNK = LPW // LCHUNK

NR = 4
NI = 7
LG = 3
LI = 6

_mesh = plsc.VectorSubcoreMesh(core_axis_name="c", subcore_axis_name="s")


@functools.partial(
    pl.kernel,
    out_type=jax.ShapeDtypeStruct((NP, D), jnp.float32),
    mesh=_mesh,
    scratch_types=[
        pltpu.VMEM((LPW,), jnp.int32),
        pltpu.VMEM((LPW, D), jnp.float32),
        pltpu.SemaphoreType.DMA,
    ],
)
def _sc_embed(emb_hbm, idx_hbm, out_hbm, idx_v, rows_v, sem):
    c = lax.axis_index("c")
    s = lax.axis_index("s")
    wid = s * NC + c
    base = wid * LPW
    pltpu.sync_copy(idx_hbm.at[pl.ds(base, LPW)], idx_v)
    for j in range(NLCHUNK):
        pltpu.async_copy(emb_hbm.at[idx_v.at[pl.ds(j * LCHUNK, LCHUNK)]],
                         rows_v.at[pl.ds(j * LCHUNK, LCHUNK)], sem)
    for j in range(NLCHUNK):
        pltpu.make_async_copy(emb_hbm.at[idx_v.at[pl.ds(j * LCHUNK, LCHUNK)]],
                              rows_v.at[pl.ds(j * LCHUNK, LCHUNK)], sem).wait()
    pltpu.sync_copy(rows_v, out_hbm.at[pl.ds(base, LPW)])


@functools.partial(
    pl.kernel,
    out_type=jax.ShapeDtypeStruct((NC, NP, D), jnp.float32),
    mesh=_mesh,
    scratch_types=[
        pltpu.VMEM((NI, CHUNK), jnp.int32),
        pltpu.VMEM((NI, CHUNK), jnp.int32),
        pltpu.VMEM((NR * CHUNK, D), jnp.float32),
        pltpu.SemaphoreType.DMA((NI,)),
        pltpu.SemaphoreType.DMA((NR,)),
        pltpu.SemaphoreType.DMA((NR,)),
        pltpu.VMEM_SHARED((NP, D), jnp.float32),
    ],
)
def _sc_agg(h_hbm, src_hbm, dst_hbm, zero_hbm, out_hbm,
            src_t, dst_t, rows_v, isem, gsem, ssem, agg_sh):
    c = lax.axis_index("c")
    s = lax.axis_index("s")
    wid = s * NC + c
    r0 = s * ROWS_PER_TILE
    pltpu.sync_copy(zero_hbm.at[pl.ds(r0, ROWS_PER_TILE)],
                    agg_sh.at[pl.ds(r0, ROWS_PER_TILE)])
    plsc.subcore_barrier()

    def rows_slot(r):
        return rows_v.at[pl.ds(r * CHUNK, CHUNK)]

    def fire_idx(ch):
        i = ch % NI
        pltpu.async_copy(src_hbm.at[wid, ch], src_t.at[i], isem.at[i])
        pltpu.async_copy(dst_hbm.at[wid, ch], dst_t.at[i], isem.at[i])

    def wait_idx(ch):
        i = ch % NI
        pltpu.make_async_copy(src_hbm.at[wid, ch], src_t.at[i],
                              isem.at[i]).wait()
        pltpu.make_async_copy(dst_hbm.at[wid, ch], dst_t.at[i],
                              isem.at[i]).wait()

    def fire_gather(ch):
        pltpu.async_copy(h_hbm.at[src_t.at[ch % NI]], rows_slot(ch % NR),
                         gsem.at[ch % NR])

    def wait_gather(ch):
        pltpu.make_async_copy(h_hbm.at[src_t.at[ch % NI]],
                              rows_slot(ch % NR), gsem.at[ch % NR]).wait()

    def fire_scatter(ch):
        pltpu.async_copy(rows_slot(ch % NR), agg_sh.at[dst_t.at[ch % NI]],
                         ssem.at[ch % NR], add=True)

    def wait_scatter(ch):
        pltpu.make_async_copy(rows_slot(ch % NR),
                              agg_sh.at[dst_t.at[ch % NI]],
                              ssem.at[ch % NR]).wait()

    for k in range(LI):
        fire_idx(k)
    for k in range(LG):
        wait_idx(k)
        fire_gather(k)

    @pl.loop(0, NCH)
    def _(ch):
        @pl.when(ch >= NR - LG)
        def _():
            wait_scatter(ch - (NR - LG))

        @pl.when(ch + LI < NCH)
        def _():
            fire_idx(ch + LI)

        @pl.when(ch + LG < NCH)
        def _():
            wait_idx(ch + LG)
            fire_gather(ch + LG)

        wait_gather(ch)
        fire_scatter(ch)

    for k in range(NCH - (NR - LG), NCH):
        wait_scatter(k)

    plsc.subcore_barrier()
    pltpu.sync_copy(agg_sh.at[pl.ds(r0, ROWS_PER_TILE)],
                    out_hbm.at[c, pl.ds(r0, ROWS_PER_TILE)])


BN = 5120


def _tc_layer_body(agg_ref, w_ref, b_ref, out_ref):
    x = agg_ref[0] + agg_ref[1]
    y = jnp.dot(x, w_ref[...], preferred_element_type=jnp.float32)
    out_ref[...] = jnp.maximum(y + b_ref[...], 0.0)


def _tc_layer_res_body(agg_ref, w_ref, b_ref, res_ref, out_ref):
    x = agg_ref[0] + agg_ref[1]
    y = jnp.dot(x, w_ref[...], preferred_element_type=jnp.float32)
    out_ref[...] = jnp.maximum(y + b_ref[...], 0.0) + res_ref[...]


def _tc_decode_body(agg_ref, w_ref, b_ref, wd_ref, bd_ref, out_ref):
    x = agg_ref[0] + agg_ref[1]
    h = jnp.maximum(
        jnp.dot(x, w_ref[...], preferred_element_type=jnp.float32) + b_ref[...],
        0.0)
    logits = jnp.dot(h, wd_ref[...], preferred_element_type=jnp.float32)
    logits = logits + bd_ref[...]
    col = lax.broadcasted_iota(jnp.int32, logits.shape, 1)
    mask = col < C
    lm = jnp.where(mask, logits, -1e30)
    m = jnp.max(lm, axis=1, keepdims=True)
    e = jnp.exp(lm - m) * mask.astype(jnp.float32)
    out_ref[...] = e / jnp.sum(e, axis=1, keepdims=True)


_agg_spec = pl.BlockSpec((NC, BN, D), lambda i: (0, i, 0))
_w_spec = pl.BlockSpec((D, D), lambda i: (0, 0))
_b_spec = pl.BlockSpec((1, D), lambda i: (0, 0))
_row_spec = pl.BlockSpec((BN, D), lambda i: (i, 0))

_tc_layer = pl.pallas_call(
    _tc_layer_body,
    grid=(NP // BN,),
    in_specs=[_agg_spec, _w_spec, _b_spec],
    out_specs=_row_spec,
    out_shape=jax.ShapeDtypeStruct((NP, D), jnp.float32),
)

_tc_layer_res = pl.pallas_call(
    _tc_layer_res_body,
    grid=(NP // BN,),
    in_specs=[_agg_spec, _w_spec, _b_spec, _row_spec],
    out_specs=_row_spec,
    out_shape=jax.ShapeDtypeStruct((NP, D), jnp.float32),
)

_tc_decode = pl.pallas_call(
    _tc_decode_body,
    grid=(NP // BN,),
    in_specs=[_agg_spec, _w_spec, _b_spec, _w_spec, _b_spec],
    out_specs=_row_spec,
    out_shape=jax.ShapeDtypeStruct((NP, D), jnp.float32),
)


def kernel(labels, edge_index, emb, W1, b1, W2, b2, W3, b3, W4, b4,
           W5, b5, Wd, bd):
    labels = labels.astype(jnp.int32)
    src = edge_index[0].astype(jnp.int32).reshape(NW, NCH, CHUNK)
    dst = edge_index[1].astype(jnp.int32).reshape(NW, NCH, CHUNK)
    labels_pad = jnp.pad(labels, (0, NP - N))
    zeros = jnp.zeros((NP, D), jnp.float32)
    b1r, b2r, b3r, b4r, b5r = (b.reshape(1, D) for b in (b1, b2, b3, b4, b5))
    wd_pad = jnp.zeros((D, D), jnp.float32).at[:, :C].set(Wd)
    bd_pad = jnp.zeros((1, D), jnp.float32).at[0, :C].set(bd)

    h0 = _sc_embed(emb, labels_pad)
    agg = _sc_agg(h0, src, dst, zeros)
    h1 = _tc_layer(agg, W1, b1r)
    agg = _sc_agg(h1, src, dst, zeros)
    h2 = _tc_layer_res(agg, W2, b2r, h1)
    agg = _sc_agg(h2, src, dst, zeros)
    h3 = _tc_layer(agg, W3, b3r)
    agg = _sc_agg(h3, src, dst, zeros)
    h4 = _tc_layer_res(agg, W4, b4r, h3)
    agg = _sc_agg(h4, src, dst, zeros)
    pred_pad = _tc_decode(agg, W5, b5r, wd_pad, bd_pad)
    return pred_pad[:N, :C]


# speedup vs baseline: 1.1402x; 1.0290x over previous
import functools

import jax
import jax.numpy as jnp
from jax import lax
from jax.experimental import pallas as pl
from jax.experimental.pallas import tpu as pltpu
from jax.experimental.pallas import tpu_sc as plsc

N = 10000
E = 320000
D = 128
V = 1000
C = 3

NP = 10240

NC = 2
NS = 16
NW = NC * NS
EPW = E // NW
CHUNK = 80
NCH = EPW // CHUNK
ROWS_PER_TILE = NP // NS

LPW = NP // NW
LCHUNK = 80
NLCHUNK = LPW // LCHUNK

NR = 4
NI = 7
LG = 3
LI = 6

_mesh = plsc.VectorSubcoreMesh(core_axis_name="c", subcore_axis_name="s")


@functools.partial(
    pl.kernel,
    out_type=jax.ShapeDtypeStruct((NP, D), jnp.float32),
    mesh=_mesh,
    scratch_types=[
        pltpu.VMEM((LPW,), jnp.int32),
        pltpu.VMEM((LPW, D), jnp.float32),
        pltpu.SemaphoreType.DMA,
    ],
)
def _sc_embed(emb_hbm, idx_hbm, out_hbm, idx_v, rows_v, sem):
    c = lax.axis_index("c")
    s = lax.axis_index("s")
    wid = s * NC + c
    base = wid * LPW
    pltpu.sync_copy(idx_hbm.at[pl.ds(base, LPW)], idx_v)
    for j in range(NLCHUNK):
        pltpu.async_copy(emb_hbm.at[idx_v.at[pl.ds(j * LCHUNK, LCHUNK)]],
                         rows_v.at[pl.ds(j * LCHUNK, LCHUNK)], sem)
    for j in range(NLCHUNK):
        pltpu.make_async_copy(emb_hbm.at[idx_v.at[pl.ds(j * LCHUNK, LCHUNK)]],
                              rows_v.at[pl.ds(j * LCHUNK, LCHUNK)], sem).wait()
    pltpu.sync_copy(rows_v, out_hbm.at[pl.ds(base, LPW)])


@functools.partial(
    pl.kernel,
    out_type=jax.ShapeDtypeStruct((NC, NP, D), jnp.float32),
    mesh=_mesh,
    scratch_types=[
        pltpu.VMEM((NI, CHUNK), jnp.int32),
        pltpu.VMEM((NI, CHUNK), jnp.int32),
        pltpu.VMEM((NR * CHUNK, D), jnp.float32),
        pltpu.SemaphoreType.DMA((NI,)),
        pltpu.SemaphoreType.DMA((NR,)),
        pltpu.SemaphoreType.DMA((NR,)),
        pltpu.SemaphoreType.DMA,
        pltpu.VMEM_SHARED((NP, D), jnp.float32),
    ],
)
def _sc_agg(h_hbm, src_hbm, dst_hbm, zero_hbm, out_hbm,
            src_t, dst_t, rows_v, isem, gsem, ssem, zsem, agg_sh):
    c = lax.axis_index("c")
    s = lax.axis_index("s")
    wid = s * NC + c
    r0 = s * ROWS_PER_TILE

    def rows_slot(r):
        return rows_v.at[pl.ds(r * CHUNK, CHUNK)]

    def fire_idx(ch):
        i = ch % NI
        pltpu.async_copy(src_hbm.at[wid, ch], src_t.at[i], isem.at[i])
        pltpu.async_copy(dst_hbm.at[wid, ch], dst_t.at[i], isem.at[i])

    def wait_idx(ch):
        i = ch % NI
        pltpu.make_async_copy(src_hbm.at[wid, ch], src_t.at[i],
                              isem.at[i]).wait()
        pltpu.make_async_copy(dst_hbm.at[wid, ch], dst_t.at[i],
                              isem.at[i]).wait()

    def fire_gather(ch):
        pltpu.async_copy(h_hbm.at[src_t.at[ch % NI]], rows_slot(ch % NR),
                         gsem.at[ch % NR])

    def wait_gather(ch):
        pltpu.make_async_copy(h_hbm.at[src_t.at[ch % NI]],
                              rows_slot(ch % NR), gsem.at[ch % NR]).wait()

    def fire_scatter(ch):
        pltpu.async_copy(rows_slot(ch % NR), agg_sh.at[dst_t.at[ch % NI]],
                         ssem.at[ch % NR], add=True)

    def wait_scatter(ch):
        pltpu.make_async_copy(rows_slot(ch % NR),
                              agg_sh.at[dst_t.at[ch % NI]],
                              ssem.at[ch % NR]).wait()

    zdesc = pltpu.make_async_copy(zero_hbm.at[pl.ds(r0, ROWS_PER_TILE)],
                                  agg_sh.at[pl.ds(r0, ROWS_PER_TILE)],
                                  zsem)
    zdesc.start()
    for k in range(LI):
        fire_idx(k)
    for k in range(LG):
        wait_idx(k)
        fire_gather(k)
    zdesc.wait()
    plsc.subcore_barrier()

    @pl.loop(0, NCH)
    def _(ch):
        @pl.when(ch >= NR - LG)
        def _():
            wait_scatter(ch - (NR - LG))

        @pl.when(ch + LI < NCH)
        def _():
            fire_idx(ch + LI)

        @pl.when(ch + LG < NCH)
        def _():
            wait_idx(ch + LG)
            fire_gather(ch + LG)

        wait_gather(ch)
        fire_scatter(ch)

    for k in range(NCH - (NR - LG), NCH):
        wait_scatter(k)

    plsc.subcore_barrier()
    pltpu.sync_copy(agg_sh.at[pl.ds(r0, ROWS_PER_TILE)],
                    out_hbm.at[c, pl.ds(r0, ROWS_PER_TILE)])


BN = 5120


def _tc_layer_body(agg_ref, w_ref, b_ref, out_ref):
    x = agg_ref[0] + agg_ref[1]
    y = jnp.dot(x, w_ref[...], preferred_element_type=jnp.float32)
    out_ref[...] = jnp.maximum(y + b_ref[...], 0.0)


def _tc_layer_res_body(agg_ref, w_ref, b_ref, res_ref, out_ref):
    x = agg_ref[0] + agg_ref[1]
    y = jnp.dot(x, w_ref[...], preferred_element_type=jnp.float32)
    out_ref[...] = jnp.maximum(y + b_ref[...], 0.0) + res_ref[...]


def _tc_decode_body(agg_ref, w_ref, b_ref, wd_ref, bd_ref, out_ref):
    x = agg_ref[0] + agg_ref[1]
    h = jnp.maximum(
        jnp.dot(x, w_ref[...], preferred_element_type=jnp.float32) + b_ref[...],
        0.0)
    logits = jnp.dot(h, wd_ref[...], preferred_element_type=jnp.float32)
    logits = logits + bd_ref[...]
    col = lax.broadcasted_iota(jnp.int32, logits.shape, 1)
    mask = col < C
    lm = jnp.where(mask, logits, -1e30)
    m = jnp.max(lm, axis=1, keepdims=True)
    e = jnp.exp(lm - m) * mask.astype(jnp.float32)
    out_ref[...] = e / jnp.sum(e, axis=1, keepdims=True)


_agg_spec = pl.BlockSpec((NC, BN, D), lambda i: (0, i, 0))
_w_spec = pl.BlockSpec((D, D), lambda i: (0, 0))
_b_spec = pl.BlockSpec((1, D), lambda i: (0, 0))
_row_spec = pl.BlockSpec((BN, D), lambda i: (i, 0))

_tc_layer = pl.pallas_call(
    _tc_layer_body,
    grid=(NP // BN,),
    in_specs=[_agg_spec, _w_spec, _b_spec],
    out_specs=_row_spec,
    out_shape=jax.ShapeDtypeStruct((NP, D), jnp.float32),
)

_tc_layer_res = pl.pallas_call(
    _tc_layer_res_body,
    grid=(NP // BN,),
    in_specs=[_agg_spec, _w_spec, _b_spec, _row_spec],
    out_specs=_row_spec,
    out_shape=jax.ShapeDtypeStruct((NP, D), jnp.float32),
)

_tc_decode = pl.pallas_call(
    _tc_decode_body,
    grid=(NP // BN,),
    in_specs=[_agg_spec, _w_spec, _b_spec, _w_spec, _b_spec],
    out_specs=_row_spec,
    out_shape=jax.ShapeDtypeStruct((NP, D), jnp.float32),
)


def kernel(labels, edge_index, emb, W1, b1, W2, b2, W3, b3, W4, b4,
           W5, b5, Wd, bd):
    labels = labels.astype(jnp.int32)
    src = edge_index[0].astype(jnp.int32).reshape(NW, NCH, CHUNK)
    dst = edge_index[1].astype(jnp.int32).reshape(NW, NCH, CHUNK)
    labels_pad = jnp.pad(labels, (0, NP - N))
    zeros = jnp.zeros((NP, D), jnp.float32)
    b1r, b2r, b3r, b4r, b5r = (b.reshape(1, D) for b in (b1, b2, b3, b4, b5))
    wd_pad = jnp.zeros((D, D), jnp.float32).at[:, :C].set(Wd)
    bd_pad = jnp.zeros((1, D), jnp.float32).at[0, :C].set(bd)

    h0 = _sc_embed(emb, labels_pad)
    agg = _sc_agg(h0, src, dst, zeros)
    h1 = _tc_layer(agg, W1, b1r)
    agg = _sc_agg(h1, src, dst, zeros)
    h2 = _tc_layer_res(agg, W2, b2r, h1)
    agg = _sc_agg(h2, src, dst, zeros)
    h3 = _tc_layer(agg, W3, b3r)
    agg = _sc_agg(h3, src, dst, zeros)
    h4 = _tc_layer_res(agg, W4, b4r, h3)
    agg = _sc_agg(h4, src, dst, zeros)
    pred_pad = _tc_decode(agg, W5, b5r, wd_pad, bd_pad)
    return pred_pad[:N, :C]
